# Initial kernel scaffold; baseline (speedup 1.0000x reference)
import jax
import jax.numpy as jnp
from jax.experimental import pallas as pl


def kernel(a, b, sign, emb_table, ctx_table):
    raise NotImplementedError("write your pallas kernel here")


# speedup vs baseline: 1.1504x; 1.1504x over previous
import functools

import jax
import jax.numpy as jnp
from jax import lax
from jax.experimental import pallas as pl
from jax.experimental.pallas import tpu as pltpu
from jax.experimental.pallas import tpu_sc as plsc

_B = 16384
_D = 128
_NC = 2
_NS = 16
_NW = _NC * _NS
_BPW = _B // _NW
_CHUNK = 128
_NCHUNK = _BPW // _CHUNK
_L = 16


_TP = _L + 1


def _sc_body(a_hbm, b_hbm, emb_hbm, ctx_hbm, out_hbm,
             aidx_v, bidx_v, arows_v, brows_v, dots_v, tbuf_v, sem_a, sem_b):
    wid = lax.axis_index("s") * _NC + lax.axis_index("c")
    base = wid * _BPW
    pltpu.sync_copy(a_hbm.at[pl.ds(base, _BPW)], aidx_v)
    pltpu.sync_copy(b_hbm.at[pl.ds(base, _BPW)], bidx_v)

    col = lax.broadcasted_iota(jnp.int32, (_L,), 0) * _TP

    for c in range(_NCHUNK):
        cp_a = pltpu.async_copy(
            emb_hbm.at[aidx_v.at[pl.ds(c * _CHUNK, _CHUNK)]], arows_v, sem_a)
        cp_b = pltpu.async_copy(
            ctx_hbm.at[bidx_v.at[pl.ds(c * _CHUNK, _CHUNK)]], brows_v, sem_b)
        cp_a.wait()
        cp_b.wait()

        def group_body(g, _, c=c):
            for j in range(_L):
                r = g * _L + j
                acc = arows_v[r, pl.ds(0, _L)] * brows_v[r, pl.ds(0, _L)]
                for k in range(1, _D // _L):
                    acc = acc + (arows_v[r, pl.ds(k * _L, _L)] *
                                 brows_v[r, pl.ds(k * _L, _L)])
                plsc.store_scatter(tbuf_v, [col + j], acc)
            vec = tbuf_v[pl.ds(0, _L)]
            for k in range(1, _L):
                vec = vec + tbuf_v[pl.ds(k * _TP, _L)]
            dots_v[pl.ds(c * _CHUNK + g * _L, _L)] = vec
            return 0

        lax.fori_loop(0, _CHUNK // _L, group_body, 0)

    pltpu.sync_copy(dots_v, out_hbm.at[pl.ds(base, _BPW)])


@functools.partial(
    pl.kernel,
    out_type=jax.ShapeDtypeStruct((_B,), jnp.float32),
    mesh=plsc.VectorSubcoreMesh(core_axis_name="c", subcore_axis_name="s"),
    compiler_params=pltpu.CompilerParams(needs_layout_passes=False),
    scratch_types=[
        pltpu.VMEM((_BPW,), jnp.int32),
        pltpu.VMEM((_BPW,), jnp.int32),
        pltpu.VMEM((_CHUNK, _D), jnp.float32),
        pltpu.VMEM((_CHUNK, _D), jnp.float32),
        pltpu.VMEM((_BPW,), jnp.float32),
        pltpu.VMEM((_L * _TP,), jnp.float32),
        pltpu.SemaphoreType.DMA,
        pltpu.SemaphoreType.DMA,
    ],
)
def _sc_dots(a_hbm, b_hbm, emb_hbm, ctx_hbm, out_hbm, *scratch):
    _sc_body(a_hbm, b_hbm, emb_hbm, ctx_hbm, out_hbm, *scratch)


def _tc_loss_body(sign_ref, dot_ref, out_ref):
    x = sign_ref[...] * dot_ref[...]
    out_ref[...] = -jax.nn.log_sigmoid(x)


_tc_loss = pl.pallas_call(
    _tc_loss_body,
    out_shape=jax.ShapeDtypeStruct((_B // _D, _D), jnp.float32),
)


def kernel(a, b, sign, emb_table, ctx_table):
    dots = _sc_dots(a, b, emb_table, ctx_table)
    loss = _tc_loss(sign.reshape(_B // _D, _D), dots.reshape(_B // _D, _D))
    return loss.reshape(_B)


# speedup vs baseline: 1.2868x; 1.1186x over previous
import functools

import jax
import jax.numpy as jnp
from jax import lax
from jax.experimental import pallas as pl
from jax.experimental.pallas import tpu as pltpu
from jax.experimental.pallas import tpu_sc as plsc

_B = 16384
_D = 128
_NC = 2
_NS = 16
_NW = _NC * _NS
_BPW = _B // _NW
_CHUNK = 128
_NCHUNK = _BPW // _CHUNK
_L = 16


_TP = _L + 1


def _sc_body(a_hbm, b_hbm, emb_hbm, ctx_hbm, out_hbm,
             aidx_v, bidx_v, arows0_v, brows0_v, arows1_v, brows1_v,
             dots_v, tbuf_v, sem_a0, sem_b0, sem_a1, sem_b1):
    wid = lax.axis_index("s") * _NC + lax.axis_index("c")
    base = wid * _BPW
    pltpu.sync_copy(a_hbm.at[pl.ds(base, _BPW)], aidx_v)
    pltpu.sync_copy(b_hbm.at[pl.ds(base, _BPW)], bidx_v)

    col = lax.broadcasted_iota(jnp.int32, (_L,), 0) * _TP
    bufs = ((arows0_v, brows0_v, sem_a0, sem_b0),
            (arows1_v, brows1_v, sem_a1, sem_b1))

    def start(c):
        ar, br, sa, sb = bufs[c % 2]
        cpa = pltpu.async_copy(
            emb_hbm.at[aidx_v.at[pl.ds(c * _CHUNK, _CHUNK)]], ar, sa)
        cpb = pltpu.async_copy(
            ctx_hbm.at[bidx_v.at[pl.ds(c * _CHUNK, _CHUNK)]], br, sb)
        return cpa, cpb

    pend = start(0)
    for c in range(_NCHUNK):
        nxt = start(c + 1) if c + 1 < _NCHUNK else None
        pend[0].wait()
        pend[1].wait()
        ar, br = bufs[c % 2][0], bufs[c % 2][1]

        def group_body(g, _, c=c, ar=ar, br=br):
            for j in range(_L):
                r = g * _L + j
                acc = ar[r, pl.ds(0, _L)] * br[r, pl.ds(0, _L)]
                for k in range(1, _D // _L):
                    acc = acc + (ar[r, pl.ds(k * _L, _L)] *
                                 br[r, pl.ds(k * _L, _L)])
                plsc.store_scatter(tbuf_v, [col + j], acc)
            vec = tbuf_v[pl.ds(0, _L)]
            for k in range(1, _L):
                vec = vec + tbuf_v[pl.ds(k * _TP, _L)]
            dots_v[pl.ds(c * _CHUNK + g * _L, _L)] = vec
            return 0

        lax.fori_loop(0, _CHUNK // _L, group_body, 0)
        pend = nxt

    pltpu.sync_copy(dots_v, out_hbm.at[pl.ds(base, _BPW)])


@functools.partial(
    pl.kernel,
    out_type=jax.ShapeDtypeStruct((_B,), jnp.float32),
    mesh=plsc.VectorSubcoreMesh(core_axis_name="c", subcore_axis_name="s"),
    compiler_params=pltpu.CompilerParams(needs_layout_passes=False),
    scratch_types=[
        pltpu.VMEM((_BPW,), jnp.int32),
        pltpu.VMEM((_BPW,), jnp.int32),
        pltpu.VMEM((_CHUNK, _D), jnp.float32),
        pltpu.VMEM((_CHUNK, _D), jnp.float32),
        pltpu.VMEM((_CHUNK, _D), jnp.float32),
        pltpu.VMEM((_CHUNK, _D), jnp.float32),
        pltpu.VMEM((_BPW,), jnp.float32),
        pltpu.VMEM((_L * _TP,), jnp.float32),
        pltpu.SemaphoreType.DMA,
        pltpu.SemaphoreType.DMA,
        pltpu.SemaphoreType.DMA,
        pltpu.SemaphoreType.DMA,
    ],
)
def _sc_dots(a_hbm, b_hbm, emb_hbm, ctx_hbm, out_hbm, *scratch):
    _sc_body(a_hbm, b_hbm, emb_hbm, ctx_hbm, out_hbm, *scratch)


def _tc_loss_body(sign_ref, dot_ref, out_ref):
    x = sign_ref[...] * dot_ref[...]
    out_ref[...] = -jax.nn.log_sigmoid(x)


_tc_loss = pl.pallas_call(
    _tc_loss_body,
    out_shape=jax.ShapeDtypeStruct((_B // _D, _D), jnp.float32),
)


def kernel(a, b, sign, emb_table, ctx_table):
    dots = _sc_dots(a, b, emb_table, ctx_table)
    loss = _tc_loss(sign.reshape(_B // _D, _D), dots.reshape(_B // _D, _D))
    return loss.reshape(_B)
